# baseline (speedup 1.0000x reference)
import functools

import jax
import jax.numpy as jnp
from jax import lax
from jax.experimental import pallas as pl
from jax.experimental.pallas import tpu as pltpu
from jax.experimental.pallas import tpu_sc as plsc

B = 16384
D = 32
H1 = 128
H2 = 64

NC = 2
NS = 16
NW = NC * NS
BPW = B // NW
CH = 128
NCH = BPW // CH

BLK = 2048


def _gather4(uid2d, mid2d, ue_t, me_t, uemf_t, memf_t):
    mesh = plsc.VectorSubcoreMesh(core_axis_name="c", subcore_axis_name="s")
    out_t = [jax.ShapeDtypeStruct((B, D), jnp.float32) for _ in range(4)]

    @functools.partial(
        pl.kernel,
        mesh=mesh,
        out_type=out_t,
        compiler_params=pltpu.CompilerParams(use_tc_tiling_on_sc=False),
        scratch_types=[
            pltpu.VMEM((NCH, CH), jnp.int32),
            pltpu.VMEM((NCH, CH), jnp.int32),
            pltpu.VMEM((BPW, D), jnp.float32),
            pltpu.VMEM((BPW, D), jnp.float32),
            pltpu.VMEM((BPW, D), jnp.float32),
            pltpu.VMEM((BPW, D), jnp.float32),
            pltpu.SemaphoreType.DMA,
        ],
    )
    def k(uid_hbm, mid_hbm, ue_hbm, me_hbm, uemf_hbm, memf_hbm,
          oue, ome, ouemf, omemf, uidx, midx, r0, r1, r2, r3, sem):
        wid = lax.axis_index("s") * NC + lax.axis_index("c")
        rbase = wid * NCH
        pltpu.sync_copy(uid_hbm.at[pl.ds(rbase, NCH)], uidx)
        pltpu.sync_copy(mid_hbm.at[pl.ds(rbase, NCH)], midx)
        copies = []
        for j in range(NCH):
            dst = pl.ds(j * CH, CH)
            copies.append(pltpu.async_copy(ue_hbm.at[uidx.at[j]], r0.at[dst], sem))
            copies.append(pltpu.async_copy(me_hbm.at[midx.at[j]], r1.at[dst], sem))
            copies.append(pltpu.async_copy(uemf_hbm.at[uidx.at[j]], r2.at[dst], sem))
            copies.append(pltpu.async_copy(memf_hbm.at[midx.at[j]], r3.at[dst], sem))
        for c in copies:
            c.wait()
        base = wid * BPW
        pltpu.sync_copy(r0, oue.at[pl.ds(base, BPW)])
        pltpu.sync_copy(r1, ome.at[pl.ds(base, BPW)])
        pltpu.sync_copy(r2, ouemf.at[pl.ds(base, BPW)])
        pltpu.sync_copy(r3, omemf.at[pl.ds(base, BPW)])

    return k(uid2d, mid2d, ue_t, me_t, uemf_t, memf_t)


def _mlp_body(ue, me, umf, mmf, w1u, w1m, b1, w2, b2, wm, wf, c, o):
    h1 = jnp.dot(ue[...], w1u[...], preferred_element_type=jnp.float32)
    h1 += jnp.dot(me[...], w1m[...], preferred_element_type=jnp.float32)
    h1 = jnp.maximum(h1 + b1[...], 0.0)
    h2 = jnp.dot(h1, w2[...], preferred_element_type=jnp.float32)
    h2 = jnp.maximum(h2 + b2[...], 0.0)
    mlp = jnp.sum(h2 * wm[...], axis=1)
    mf = jnp.sum((umf[...] * mmf[...]) * wf[...], axis=1)
    o[...] = mlp + mf + c[0]


def _mlp(ue, me, uemf, memf, w1u, w1m, b1r, w2, b2r, wm, wf, c):
    grid = (B // BLK,)
    row_spec = pl.BlockSpec((BLK, D), lambda i: (i, 0))
    fixed = lambda shape: pl.BlockSpec(shape, lambda i: (0, 0))
    return pl.pallas_call(
        _mlp_body,
        grid=grid,
        in_specs=[
            row_spec, row_spec, row_spec, row_spec,
            fixed((D, H1)), fixed((D, H1)), fixed((1, H1)),
            fixed((H1, H2)), fixed((1, H2)),
            fixed((1, H2)), fixed((1, D)),
            pl.BlockSpec(memory_space=pltpu.SMEM),
        ],
        out_specs=pl.BlockSpec((BLK,), lambda i: (i,)),
        out_shape=jax.ShapeDtypeStruct((B,), jnp.float32),
    )(ue, me, uemf, memf, w1u, w1m, b1r, w2, b2r, wm, wf, c)


def kernel(user_ids, movie_ids, user_emb, movie_emb, user_emb_mf, movie_emb_mf,
           W1, b1, W2, b2, W_mlp_out, b_mlp_out, W_mf, b_mf, W_final, b_final):
    uid2d = user_ids.astype(jnp.int32).reshape(B // CH, CH)
    mid2d = movie_ids.astype(jnp.int32).reshape(B // CH, CH)

    ue, me, uemf, memf = _gather4(uid2d, mid2d, user_emb, movie_emb,
                                  user_emb_mf, movie_emb_mf)

    wf0 = W_final[0, 0]
    wf1 = W_final[1, 0]
    wm = (W_mlp_out[:, 0] * wf0).reshape(1, H2)
    wf = (W_mf[:, 0] * wf1).reshape(1, D)
    c = (b_mlp_out[0] * wf0 + b_mf[0] * wf1 + b_final[0]).reshape(1)

    return _mlp(ue, me, uemf, memf,
                W1[:D], W1[D:], b1.reshape(1, H1),
                W2, b2.reshape(1, H2), wm, wf, c)
